# baseline (speedup 1.0000x reference)
import functools

import jax
import jax.numpy as jnp
from jax import lax
from jax.experimental import pallas as pl
from jax.experimental.pallas import tpu as pltpu
from jax.experimental.pallas import tpu_sc as plsc

NUM_EXPERTS = 8
TOP_K = 2
D_MODEL = 768
D_FF = 2048
SEQ = 2048

BT = 128
NPAIR = SEQ * TOP_K
NPAD = 5120
NBLK = NPAD // BT
NW = 32
PPW = NPAIR // NW
TPW = SEQ // NW

def _sc_mesh():
    return plsc.VectorSubcoreMesh(
        core_axis_name="c", subcore_axis_name="s", num_cores=2, num_subcores=16)


def _router_body(x_ref, wr_ref, rwa_ref, rwb_ref, slots_ref, cnts_ref):
    logits = jnp.dot(x_ref[...], wr_ref[...], preferred_element_type=jnp.float32)
    T = logits.shape[0]
    lane = lax.broadcasted_iota(jnp.int32, logits.shape, 1)
    i1 = jnp.argmax(logits, axis=1, keepdims=True)
    oh1 = lane == i1
    m1 = jnp.max(logits, axis=1, keepdims=True)
    masked = jnp.where(oh1, -jnp.inf, logits)
    i2 = jnp.argmax(masked, axis=1, keepdims=True)
    oh2 = lane == i2
    m2 = jnp.max(masked, axis=1, keepdims=True)
    w1 = 1.0 / (1.0 + jnp.exp(m2 - m1))
    w2 = 1.0 / (1.0 + jnp.exp(m1 - m2))

    C = oh1.astype(jnp.float32) + oh2.astype(jnp.float32)
    CH = 128
    r_i = lax.broadcasted_iota(jnp.int32, (CH, CH), 0)
    c_i = lax.broadcasted_iota(jnp.int32, (CH, CH), 1)
    tri = (c_i < r_i).astype(jnp.float32)
    off = jnp.zeros((1, NUM_EXPERTS), jnp.float32)
    parts = []
    for k in range(T // CH):
        Ck = C[k * CH:(k + 1) * CH]
        parts.append(jnp.dot(tri, Ck, preferred_element_type=jnp.float32) + off)
        off = off + jnp.sum(Ck, axis=0, keepdims=True)
    S = jnp.concatenate(parts, axis=0)

    cnts = off
    padded = jnp.ceil(cnts / BT) * BT
    a_i = lax.broadcasted_iota(jnp.int32, (NUM_EXPERTS, NUM_EXPERTS), 0)
    b_i = lax.broadcasted_iota(jnp.int32, (NUM_EXPERTS, NUM_EXPERTS), 1)
    upper = (a_i < b_i).astype(jnp.float32)
    start = jnp.dot(padded, upper, preferred_element_type=jnp.float32)

    s1 = jnp.sum(jnp.where(oh1, S, 0.0), axis=1, keepdims=True)
    s2 = jnp.sum(jnp.where(oh2, S, 0.0), axis=1, keepdims=True)
    st1 = jnp.sum(jnp.where(oh1, start, 0.0), axis=1, keepdims=True)
    st2 = jnp.sum(jnp.where(oh2, start, 0.0), axis=1, keepdims=True)
    slot1 = (s1 + st1).astype(jnp.int32)
    slot2 = (s2 + st2).astype(jnp.int32)

    slots_ref[...] = jnp.concatenate([slot1, slot2], axis=1)
    rwa_ref[...] = jnp.broadcast_to(w1, (T, 128))
    rwb_ref[...] = jnp.broadcast_to(w2, (T, 128))
    cnts_ref[...] = cnts


def _dispatch_body(flat_hbm, sa_hbm, sb_hbm, wa_hbm, wb_hbm, xs_hbm, wgt_hbm,
                   rows_v, sa_v, sb_v, wa_v, wb_v, s0, s1, s2, s3):
    wid = lax.axis_index("s") * 2 + lax.axis_index("c")
    t0 = wid * TPW
    pltpu.sync_copy(sa_hbm.at[pl.ds(t0, TPW)], sa_v.at[0])
    pltpu.sync_copy(sb_hbm.at[pl.ds(t0, TPW)], sb_v.at[0])
    pltpu.sync_copy(flat_hbm.at[pl.ds(t0, TPW)], rows_v)
    ca = pltpu.async_copy(rows_v, xs_hbm.at[sa_v.at[0]], s0)
    cb = pltpu.async_copy(rows_v, xs_hbm.at[sb_v.at[0]], s1)
    pltpu.sync_copy(wa_hbm.at[pl.ds(t0, TPW)], wa_v)
    pltpu.sync_copy(wb_hbm.at[pl.ds(t0, TPW)], wb_v)
    cc = pltpu.async_copy(wa_v, wgt_hbm.at[sa_v.at[0]], s2)
    cd = pltpu.async_copy(wb_v, wgt_hbm.at[sb_v.at[0]], s3)
    ca.wait()
    cb.wait()
    cc.wait()
    cd.wait()


def _moe_body(be_ref, gi_ref, fetch_ref, nx_ref, nact_ref,
              xs_ref, w1_hbm, w3_hbm, w2_hbm, wgt_ref, out_ref,
              w1b, w3b, w2b, sems):
    i = pl.program_id(0)
    e = be_ref[i]
    g = gi_ref[i]
    p = lax.rem(g, 2)
    prev_g = gi_ref[jnp.maximum(i - 1, 0)]
    first = jnp.logical_or(i == 0, prev_g != g)

    @pl.when(i == 0)
    def _():
        pltpu.make_async_copy(w1_hbm.at[e], w1b.at[0], sems.at[0, 0]).start()
        pltpu.make_async_copy(w3_hbm.at[e], w3b.at[0], sems.at[0, 1]).start()
        pltpu.make_async_copy(w2_hbm.at[e], w2b.at[0], sems.at[0, 2]).start()

    @pl.when(jnp.logical_and(first, fetch_ref[i] == 1))
    def _():
        ne = nx_ref[i]
        q = lax.rem(g + 1, 2)
        pltpu.make_async_copy(w1_hbm.at[ne], w1b.at[q], sems.at[q, 0]).start()
        pltpu.make_async_copy(w3_hbm.at[ne], w3b.at[q], sems.at[q, 1]).start()
        pltpu.make_async_copy(w2_hbm.at[ne], w2b.at[q], sems.at[q, 2]).start()

    @pl.when(first)
    def _():
        pltpu.make_async_copy(w1_hbm.at[e], w1b.at[p], sems.at[p, 0]).wait()
        pltpu.make_async_copy(w3_hbm.at[e], w3b.at[p], sems.at[p, 1]).wait()
        pltpu.make_async_copy(w2_hbm.at[e], w2b.at[p], sems.at[p, 2]).wait()

    @pl.when(i < nact_ref[0])
    def _():
        x = xs_ref[...]
        a = jnp.dot(x, w1b[p], preferred_element_type=jnp.float32)
        b = jnp.dot(x, w3b[p], preferred_element_type=jnp.float32)
        h = (a / (1.0 + jnp.exp(-a))) * b
        y = jnp.dot(h, w2b[p], preferred_element_type=jnp.float32)
        out_ref[...] = y * wgt_ref[...][:, 0:1]


def _combine_body(yw_hbm, slots_hbm, out_hbm, slot_v, yw_v, out_v, sem):
    wid = lax.axis_index("s") * 2 + lax.axis_index("c")
    for c in range(2):
        basep = wid * PPW + 64 * c
        pltpu.sync_copy(slots_hbm.at[pl.ds(basep, 64)], slot_v.at[0])
        pltpu.async_copy(yw_hbm.at[slot_v.at[0]], yw_v, sem).wait()

        def body_i(i, carry):
            for j in range(D_MODEL // 16):
                s = pl.ds(j * 16, 16)
                out_v[i, s] = yw_v[2 * i, s] + yw_v[2 * i + 1, s]
            return carry

        lax.fori_loop(0, 32, body_i, 0)
        pltpu.sync_copy(out_v, out_hbm.at[pl.ds(wid * TPW + 32 * c, 32)])


def _dispatch_call(flat, sa, sb, wa, wb):
    fn = pl.kernel(
        _dispatch_body,
        out_type=[jax.ShapeDtypeStruct((NPAD, D_MODEL), jnp.float32),
                  jax.ShapeDtypeStruct((NPAD, 128), jnp.float32)],
        mesh=_sc_mesh(),
        scratch_types=[pltpu.VMEM((TPW, D_MODEL), jnp.float32),
                       pltpu.VMEM((1, TPW), jnp.int32),
                       pltpu.VMEM((1, TPW), jnp.int32),
                       pltpu.VMEM((TPW, 128), jnp.float32),
                       pltpu.VMEM((TPW, 128), jnp.float32),
                       pltpu.SemaphoreType.DMA,
                       pltpu.SemaphoreType.DMA,
                       pltpu.SemaphoreType.DMA,
                       pltpu.SemaphoreType.DMA],
    )
    return fn(flat, sa, sb, wa, wb)


def _combine_call(yw, slots_flat):
    fn = pl.kernel(
        _combine_body,
        out_type=jax.ShapeDtypeStruct((SEQ, D_MODEL), jnp.float32),
        mesh=_sc_mesh(),
        scratch_types=[pltpu.VMEM((1, 64), jnp.int32),
                       pltpu.VMEM((64, D_MODEL), jnp.float32),
                       pltpu.VMEM((32, D_MODEL), jnp.float32),
                       pltpu.SemaphoreType.DMA],
    )
    return fn(yw, slots_flat)


def kernel(hidden_states, Wr, W1, W2, W3):
    batch, seq, hidden = hidden_states.shape
    flat = hidden_states.reshape(-1, hidden)

    rwa, rwb, slots2, cnts = pl.pallas_call(
        _router_body,
        out_shape=[jax.ShapeDtypeStruct((SEQ, 128), jnp.float32),
                   jax.ShapeDtypeStruct((SEQ, 128), jnp.float32),
                   jax.ShapeDtypeStruct((SEQ, TOP_K), jnp.int32),
                   jax.ShapeDtypeStruct((1, NUM_EXPERTS), jnp.float32)],
    )(flat, Wr)

    slots_flat = slots2.reshape(NPAIR)
    sa = slots2[:, 0]
    sb = slots2[:, 1]

    padded = (jnp.ceil(cnts[0] / BT) * BT).astype(jnp.int32)
    pstart = (jnp.concatenate([jnp.zeros((1,), jnp.int32),
                               jnp.cumsum(padded)[:-1]]) // BT)
    be = jnp.clip(
        (jnp.arange(NBLK, dtype=jnp.int32)[:, None] >= pstart[None, :])
        .sum(axis=1).astype(jnp.int32) - 1, 0, NUM_EXPERTS - 1)
    first = jnp.concatenate([jnp.ones((1,), jnp.int32),
                             (be[1:] != be[:-1]).astype(jnp.int32)])
    gi = jnp.cumsum(first) - 1
    ngroups = gi[-1] + 1
    gexp = jnp.zeros((NBLK + 1,), jnp.int32).at[gi].set(be)
    nx = gexp[jnp.clip(gi + 1, 0, NBLK)]
    fetch = (first * (gi + 1 < ngroups)).astype(jnp.int32)
    nact = (jnp.sum(padded) // BT).astype(jnp.int32).reshape(1)

    xs, wgt = _dispatch_call(flat, sa, sb, rwa, rwb)

    yw = pl.pallas_call(
        _moe_body,
        grid_spec=pltpu.PrefetchScalarGridSpec(
            num_scalar_prefetch=5,
            grid=(NBLK,),
            in_specs=[
                pl.BlockSpec((BT, D_MODEL), lambda i, *_: (i, 0)),
                pl.BlockSpec(memory_space=pl.ANY),
                pl.BlockSpec(memory_space=pl.ANY),
                pl.BlockSpec(memory_space=pl.ANY),
                pl.BlockSpec((BT, 128), lambda i, *_: (i, 0)),
            ],
            out_specs=pl.BlockSpec((BT, D_MODEL), lambda i, *_: (i, 0)),
            scratch_shapes=[
                pltpu.VMEM((2, D_MODEL, D_FF), jnp.float32),
                pltpu.VMEM((2, D_MODEL, D_FF), jnp.float32),
                pltpu.VMEM((2, D_FF, D_MODEL), jnp.float32),
                pltpu.SemaphoreType.DMA((2, 3)),
            ],
        ),
        out_shape=jax.ShapeDtypeStruct((NPAD, D_MODEL), jnp.float32),
    )(be, gi.astype(jnp.int32), fetch, nx.astype(jnp.int32), nact,
      xs, W1, W3, W2, wgt)

    out = _combine_call(yw, slots_flat)
    return out.reshape(batch, seq, hidden)


# speedup vs baseline: 1.0048x; 1.0048x over previous
import functools

import jax
import jax.numpy as jnp
from jax import lax
from jax.experimental import pallas as pl
from jax.experimental.pallas import tpu as pltpu
from jax.experimental.pallas import tpu_sc as plsc

NUM_EXPERTS = 8
TOP_K = 2
D_MODEL = 768
D_FF = 2048
SEQ = 2048

BT = 128
NPAIR = SEQ * TOP_K
NPAD = 5120
NBLK = NPAD // BT
NW = 32
PPW = NPAIR // NW
TPW = SEQ // NW

def _sc_mesh():
    return plsc.VectorSubcoreMesh(
        core_axis_name="c", subcore_axis_name="s", num_cores=2, num_subcores=16)


def _router_body(x_ref, wr_ref, rwa_ref, rwb_ref, slots_ref, cnts_ref):
    logits = jnp.dot(x_ref[...], wr_ref[...], preferred_element_type=jnp.float32)
    T = logits.shape[0]
    lane = lax.broadcasted_iota(jnp.int32, logits.shape, 1)
    i1 = jnp.argmax(logits, axis=1, keepdims=True)
    oh1 = lane == i1
    m1 = jnp.max(logits, axis=1, keepdims=True)
    masked = jnp.where(oh1, -jnp.inf, logits)
    i2 = jnp.argmax(masked, axis=1, keepdims=True)
    oh2 = lane == i2
    m2 = jnp.max(masked, axis=1, keepdims=True)
    w1 = 1.0 / (1.0 + jnp.exp(m2 - m1))
    w2 = 1.0 / (1.0 + jnp.exp(m1 - m2))

    C = oh1.astype(jnp.float32) + oh2.astype(jnp.float32)
    CH = 128
    r_i = lax.broadcasted_iota(jnp.int32, (CH, CH), 0)
    c_i = lax.broadcasted_iota(jnp.int32, (CH, CH), 1)
    tri = (c_i < r_i).astype(jnp.float32)
    off = jnp.zeros((1, NUM_EXPERTS), jnp.float32)
    parts = []
    for k in range(T // CH):
        Ck = C[k * CH:(k + 1) * CH]
        parts.append(jnp.dot(tri, Ck, preferred_element_type=jnp.float32) + off)
        off = off + jnp.sum(Ck, axis=0, keepdims=True)
    S = jnp.concatenate(parts, axis=0)

    cnts = off
    padded = jnp.ceil(cnts / BT) * BT
    a_i = lax.broadcasted_iota(jnp.int32, (NUM_EXPERTS, NUM_EXPERTS), 0)
    b_i = lax.broadcasted_iota(jnp.int32, (NUM_EXPERTS, NUM_EXPERTS), 1)
    upper = (a_i < b_i).astype(jnp.float32)
    start = jnp.dot(padded, upper, preferred_element_type=jnp.float32)

    s1 = jnp.sum(jnp.where(oh1, S, 0.0), axis=1, keepdims=True)
    s2 = jnp.sum(jnp.where(oh2, S, 0.0), axis=1, keepdims=True)
    st1 = jnp.sum(jnp.where(oh1, start, 0.0), axis=1, keepdims=True)
    st2 = jnp.sum(jnp.where(oh2, start, 0.0), axis=1, keepdims=True)
    slot1 = (s1 + st1).astype(jnp.int32)
    slot2 = (s2 + st2).astype(jnp.int32)

    slots_ref[...] = jnp.concatenate([slot1, slot2], axis=1)
    rwa_ref[...] = jnp.broadcast_to(w1, (T, 128))
    rwb_ref[...] = jnp.broadcast_to(w2, (T, 128))
    cnts_ref[...] = cnts


def _dispatch_body(flat_hbm, sa_hbm, sb_hbm, wa_hbm, wb_hbm, xs_hbm, wgt_hbm,
                   rows_v, sa_v, sb_v, wa_v, wb_v, s0, s1, s2, s3):
    wid = lax.axis_index("s") * 2 + lax.axis_index("c")
    t0 = wid * TPW
    pltpu.sync_copy(sa_hbm.at[pl.ds(t0, TPW)], sa_v.at[0])
    pltpu.sync_copy(sb_hbm.at[pl.ds(t0, TPW)], sb_v.at[0])
    pltpu.sync_copy(flat_hbm.at[pl.ds(t0, TPW)], rows_v)
    ca = pltpu.async_copy(rows_v, xs_hbm.at[sa_v.at[0]], s0)
    cb = pltpu.async_copy(rows_v, xs_hbm.at[sb_v.at[0]], s1)
    pltpu.sync_copy(wa_hbm.at[pl.ds(t0, TPW)], wa_v)
    pltpu.sync_copy(wb_hbm.at[pl.ds(t0, TPW)], wb_v)
    cc = pltpu.async_copy(wa_v, wgt_hbm.at[sa_v.at[0]], s2)
    cd = pltpu.async_copy(wb_v, wgt_hbm.at[sb_v.at[0]], s3)
    ca.wait()
    cb.wait()
    cc.wait()
    cd.wait()


def _moe_body(be_ref, gi_ref, fetch_ref, nx_ref, nact_ref,
              xs_ref, w1_hbm, w3_hbm, w2_hbm, wgt_ref, out_ref,
              w1b, w3b, w2b, sems):
    i = pl.program_id(0)
    e = be_ref[i]
    g = gi_ref[i]
    p = lax.rem(g, 2)
    prev_g = gi_ref[jnp.maximum(i - 1, 0)]
    first = jnp.logical_or(i == 0, prev_g != g)

    @pl.when(i == 0)
    def _():
        pltpu.make_async_copy(w1_hbm.at[e], w1b.at[0], sems.at[0, 0]).start()
        pltpu.make_async_copy(w3_hbm.at[e], w3b.at[0], sems.at[0, 1]).start()
        pltpu.make_async_copy(w2_hbm.at[e], w2b.at[0], sems.at[0, 2]).start()

    @pl.when(jnp.logical_and(first, fetch_ref[i] == 1))
    def _():
        ne = nx_ref[i]
        q = lax.rem(g + 1, 2)
        pltpu.make_async_copy(w1_hbm.at[ne], w1b.at[q], sems.at[q, 0]).start()
        pltpu.make_async_copy(w3_hbm.at[ne], w3b.at[q], sems.at[q, 1]).start()
        pltpu.make_async_copy(w2_hbm.at[ne], w2b.at[q], sems.at[q, 2]).start()

    @pl.when(first)
    def _():
        pltpu.make_async_copy(w1_hbm.at[e], w1b.at[p], sems.at[p, 0]).wait()
        pltpu.make_async_copy(w3_hbm.at[e], w3b.at[p], sems.at[p, 1]).wait()
        pltpu.make_async_copy(w2_hbm.at[e], w2b.at[p], sems.at[p, 2]).wait()

    @pl.when(i < nact_ref[0])
    def _():
        x = xs_ref[...]
        a = jnp.dot(x, w1b[p], preferred_element_type=jnp.float32)
        b = jnp.dot(x, w3b[p], preferred_element_type=jnp.float32)
        h = (a / (1.0 + jnp.exp(-a))) * b
        y = jnp.dot(h, w2b[p], preferred_element_type=jnp.float32)
        out_ref[...] = y * wgt_ref[...][:, 0:1]


def _combine_body(yw_hbm, slots_hbm, out_hbm,
                  slot_v, ywa_v, ywb_v, outa_v, s0, s1, s2):
    wid = lax.axis_index("s") * 2 + lax.axis_index("c")
    base = wid * PPW
    pltpu.sync_copy(slots_hbm.at[pl.ds(base, PPW)], slot_v.at[0])
    ca = pltpu.async_copy(yw_hbm.at[slot_v.at[0, pl.ds(0, 64)]], ywa_v, s0)
    cb = pltpu.async_copy(yw_hbm.at[slot_v.at[0, pl.ds(64, 64)]], ywb_v, s1)

    def add_pairs(src_v, dst_v):
        def body_i(i, carry):
            for j in range(D_MODEL // 16):
                s = pl.ds(j * 16, 16)
                dst_v[i, s] = src_v[2 * i, s] + src_v[2 * i + 1, s]
            return carry
        lax.fori_loop(0, 32, body_i, 0)

    ca.wait()
    add_pairs(ywa_v, outa_v)
    wa = pltpu.async_copy(outa_v, out_hbm.at[pl.ds(wid * TPW, 32)], s2)
    cb.wait()
    wa.wait()
    add_pairs(ywb_v, outa_v)
    pltpu.sync_copy(outa_v, out_hbm.at[pl.ds(wid * TPW + 32, 32)])


def _dispatch_call(flat, sa, sb, wa, wb):
    fn = pl.kernel(
        _dispatch_body,
        out_type=[jax.ShapeDtypeStruct((NPAD, D_MODEL), jnp.float32),
                  jax.ShapeDtypeStruct((NPAD, 128), jnp.float32)],
        mesh=_sc_mesh(),
        scratch_types=[pltpu.VMEM((TPW, D_MODEL), jnp.float32),
                       pltpu.VMEM((1, TPW), jnp.int32),
                       pltpu.VMEM((1, TPW), jnp.int32),
                       pltpu.VMEM((TPW, 128), jnp.float32),
                       pltpu.VMEM((TPW, 128), jnp.float32),
                       pltpu.SemaphoreType.DMA,
                       pltpu.SemaphoreType.DMA,
                       pltpu.SemaphoreType.DMA,
                       pltpu.SemaphoreType.DMA],
    )
    return fn(flat, sa, sb, wa, wb)


def _combine_call(yw, slots_flat):
    fn = pl.kernel(
        _combine_body,
        out_type=jax.ShapeDtypeStruct((SEQ, D_MODEL), jnp.float32),
        mesh=_sc_mesh(),
        scratch_types=[pltpu.VMEM((1, PPW), jnp.int32),
                       pltpu.VMEM((64, D_MODEL), jnp.float32),
                       pltpu.VMEM((64, D_MODEL), jnp.float32),
                       pltpu.VMEM((32, D_MODEL), jnp.float32),
                       pltpu.SemaphoreType.DMA,
                       pltpu.SemaphoreType.DMA,
                       pltpu.SemaphoreType.DMA],
    )
    return fn(yw, slots_flat)


def kernel(hidden_states, Wr, W1, W2, W3):
    batch, seq, hidden = hidden_states.shape
    flat = hidden_states.reshape(-1, hidden)

    rwa, rwb, slots2, cnts = pl.pallas_call(
        _router_body,
        out_shape=[jax.ShapeDtypeStruct((SEQ, 128), jnp.float32),
                   jax.ShapeDtypeStruct((SEQ, 128), jnp.float32),
                   jax.ShapeDtypeStruct((SEQ, TOP_K), jnp.int32),
                   jax.ShapeDtypeStruct((1, NUM_EXPERTS), jnp.float32)],
    )(flat, Wr)

    slots_flat = slots2.reshape(NPAIR)
    sa = slots2[:, 0]
    sb = slots2[:, 1]

    padded = (jnp.ceil(cnts[0] / BT) * BT).astype(jnp.int32)
    pstart = (jnp.concatenate([jnp.zeros((1,), jnp.int32),
                               jnp.cumsum(padded)[:-1]]) // BT)
    be = jnp.clip(
        (jnp.arange(NBLK, dtype=jnp.int32)[:, None] >= pstart[None, :])
        .sum(axis=1).astype(jnp.int32) - 1, 0, NUM_EXPERTS - 1)
    first = jnp.concatenate([jnp.ones((1,), jnp.int32),
                             (be[1:] != be[:-1]).astype(jnp.int32)])
    gi = jnp.cumsum(first) - 1
    ngroups = gi[-1] + 1
    gexp = jnp.zeros((NBLK + 1,), jnp.int32).at[gi].set(be)
    nx = gexp[jnp.clip(gi + 1, 0, NBLK)]
    fetch = (first * (gi + 1 < ngroups)).astype(jnp.int32)
    nact = (jnp.sum(padded) // BT).astype(jnp.int32).reshape(1)

    xs, wgt = _dispatch_call(flat, sa, sb, rwa, rwb)

    yw = pl.pallas_call(
        _moe_body,
        grid_spec=pltpu.PrefetchScalarGridSpec(
            num_scalar_prefetch=5,
            grid=(NBLK,),
            in_specs=[
                pl.BlockSpec((BT, D_MODEL), lambda i, *_: (i, 0)),
                pl.BlockSpec(memory_space=pl.ANY),
                pl.BlockSpec(memory_space=pl.ANY),
                pl.BlockSpec(memory_space=pl.ANY),
                pl.BlockSpec((BT, 128), lambda i, *_: (i, 0)),
            ],
            out_specs=pl.BlockSpec((BT, D_MODEL), lambda i, *_: (i, 0)),
            scratch_shapes=[
                pltpu.VMEM((2, D_MODEL, D_FF), jnp.float32),
                pltpu.VMEM((2, D_MODEL, D_FF), jnp.float32),
                pltpu.VMEM((2, D_FF, D_MODEL), jnp.float32),
                pltpu.SemaphoreType.DMA((2, 3)),
            ],
        ),
        out_shape=jax.ShapeDtypeStruct((NPAD, D_MODEL), jnp.float32),
    )(be, gi.astype(jnp.int32), fetch, nx.astype(jnp.int32), nact,
      xs, W1, W3, W2, wgt)

    out = _combine_call(yw, slots_flat)
    return out.reshape(batch, seq, hidden)


# speedup vs baseline: 1.0310x; 1.0261x over previous
import functools

import jax
import jax.numpy as jnp
from jax import lax
from jax.experimental import pallas as pl
from jax.experimental.pallas import tpu as pltpu
from jax.experimental.pallas import tpu_sc as plsc

NUM_EXPERTS = 8
TOP_K = 2
D_MODEL = 768
D_FF = 2048
SEQ = 2048

BT = 128
NPAIR = SEQ * TOP_K
NPAD = 5120
NBLK = NPAD // BT
NW = 32
PPW = NPAIR // NW
TPW = SEQ // NW

def _sc_mesh():
    return plsc.VectorSubcoreMesh(
        core_axis_name="c", subcore_axis_name="s", num_cores=2, num_subcores=16)


def _router_body(x_ref, wr_ref, rwa_ref, rwb_ref, slots_ref, cnts_ref):
    logits = jnp.dot(x_ref[...], wr_ref[...], preferred_element_type=jnp.float32)
    T = logits.shape[0]
    lane = lax.broadcasted_iota(jnp.int32, logits.shape, 1)
    i1 = jnp.argmax(logits, axis=1, keepdims=True)
    oh1 = lane == i1
    m1 = jnp.max(logits, axis=1, keepdims=True)
    masked = jnp.where(oh1, -jnp.inf, logits)
    i2 = jnp.argmax(masked, axis=1, keepdims=True)
    oh2 = lane == i2
    m2 = jnp.max(masked, axis=1, keepdims=True)
    w1 = 1.0 / (1.0 + jnp.exp(m2 - m1))
    w2 = 1.0 / (1.0 + jnp.exp(m1 - m2))

    C = oh1.astype(jnp.float32) + oh2.astype(jnp.float32)
    CH = 128
    r_i = lax.broadcasted_iota(jnp.int32, (CH, CH), 0)
    c_i = lax.broadcasted_iota(jnp.int32, (CH, CH), 1)
    tri = (c_i < r_i).astype(jnp.float32)
    off = jnp.zeros((1, NUM_EXPERTS), jnp.float32)
    parts = []
    for k in range(T // CH):
        Ck = C[k * CH:(k + 1) * CH]
        parts.append(jnp.dot(tri, Ck, preferred_element_type=jnp.float32) + off)
        off = off + jnp.sum(Ck, axis=0, keepdims=True)
    S = jnp.concatenate(parts, axis=0)

    cnts = off
    padded = jnp.ceil(cnts / BT) * BT
    a_i = lax.broadcasted_iota(jnp.int32, (NUM_EXPERTS, NUM_EXPERTS), 0)
    b_i = lax.broadcasted_iota(jnp.int32, (NUM_EXPERTS, NUM_EXPERTS), 1)
    upper = (a_i < b_i).astype(jnp.float32)
    start = jnp.dot(padded, upper, preferred_element_type=jnp.float32)

    s1 = jnp.sum(jnp.where(oh1, S, 0.0), axis=1, keepdims=True)
    s2 = jnp.sum(jnp.where(oh2, S, 0.0), axis=1, keepdims=True)
    st1 = jnp.sum(jnp.where(oh1, start, 0.0), axis=1, keepdims=True)
    st2 = jnp.sum(jnp.where(oh2, start, 0.0), axis=1, keepdims=True)
    slot1 = (s1 + st1).astype(jnp.int32)
    slot2 = (s2 + st2).astype(jnp.int32)

    slots_ref[...] = jnp.concatenate([slot1, slot2], axis=1)
    rwa_ref[...] = jnp.broadcast_to(w1, (T, 128))
    rwb_ref[...] = jnp.broadcast_to(w2, (T, 128))
    cnts_ref[...] = cnts


def _dispatch_body(flat_hbm, sa_hbm, sb_hbm, xs_hbm,
                   rows_v, sa_v, sb_v, s0, s1):
    wid = lax.axis_index("s") * 2 + lax.axis_index("c")
    t0 = wid * TPW
    pltpu.sync_copy(sa_hbm.at[pl.ds(t0, TPW)], sa_v.at[0])
    pltpu.sync_copy(sb_hbm.at[pl.ds(t0, TPW)], sb_v.at[0])
    pltpu.sync_copy(flat_hbm.at[pl.ds(t0, TPW)], rows_v)
    ca = pltpu.async_copy(rows_v, xs_hbm.at[sa_v.at[0]], s0)
    cb = pltpu.async_copy(rows_v, xs_hbm.at[sb_v.at[0]], s1)
    ca.wait()
    cb.wait()


def _moe_body(be_ref, gi_ref, fetch_ref, nx_ref, nact_ref,
              xs_ref, w1_hbm, w3_hbm, w2_hbm, out_ref,
              w1b, w3b, w2b, sems):
    i = pl.program_id(0)
    e = be_ref[i]
    g = gi_ref[i]
    p = lax.rem(g, 2)
    prev_g = gi_ref[jnp.maximum(i - 1, 0)]
    first = jnp.logical_or(i == 0, prev_g != g)

    @pl.when(i == 0)
    def _():
        pltpu.make_async_copy(w1_hbm.at[e], w1b.at[0], sems.at[0, 0]).start()
        pltpu.make_async_copy(w3_hbm.at[e], w3b.at[0], sems.at[0, 1]).start()
        pltpu.make_async_copy(w2_hbm.at[e], w2b.at[0], sems.at[0, 2]).start()

    @pl.when(jnp.logical_and(first, fetch_ref[i] == 1))
    def _():
        ne = nx_ref[i]
        q = lax.rem(g + 1, 2)
        pltpu.make_async_copy(w1_hbm.at[ne], w1b.at[q], sems.at[q, 0]).start()
        pltpu.make_async_copy(w3_hbm.at[ne], w3b.at[q], sems.at[q, 1]).start()
        pltpu.make_async_copy(w2_hbm.at[ne], w2b.at[q], sems.at[q, 2]).start()

    @pl.when(first)
    def _():
        pltpu.make_async_copy(w1_hbm.at[e], w1b.at[p], sems.at[p, 0]).wait()
        pltpu.make_async_copy(w3_hbm.at[e], w3b.at[p], sems.at[p, 1]).wait()
        pltpu.make_async_copy(w2_hbm.at[e], w2b.at[p], sems.at[p, 2]).wait()

    @pl.when(i < nact_ref[0])
    def _():
        x = xs_ref[...]
        a = jnp.dot(x, w1b[p], preferred_element_type=jnp.float32)
        b = jnp.dot(x, w3b[p], preferred_element_type=jnp.float32)
        h = (a / (1.0 + jnp.exp(-a))) * b
        out_ref[...] = jnp.dot(h, w2b[p], preferred_element_type=jnp.float32)


def _combine_body(yw_hbm, sa_hbm, sb_hbm, ya_hbm, yb_hbm,
                  sa_v, sb_v, ra_v, rb_v, s0, s1, s2, s3):
    wid = lax.axis_index("s") * 2 + lax.axis_index("c")
    t0 = wid * TPW
    pltpu.sync_copy(sa_hbm.at[pl.ds(t0, TPW)], sa_v.at[0])
    pltpu.sync_copy(sb_hbm.at[pl.ds(t0, TPW)], sb_v.at[0])
    ca = pltpu.async_copy(yw_hbm.at[sa_v.at[0]], ra_v, s0)
    cb = pltpu.async_copy(yw_hbm.at[sb_v.at[0]], rb_v, s1)
    ca.wait()
    wa = pltpu.async_copy(ra_v, ya_hbm.at[pl.ds(t0, TPW)], s2)
    cb.wait()
    wb = pltpu.async_copy(rb_v, yb_hbm.at[pl.ds(t0, TPW)], s3)
    wa.wait()
    wb.wait()


def _finish_body(ya_ref, yb_ref, wa_ref, wb_ref, out_ref):
    out_ref[...] = (wa_ref[...][:, 0:1] * ya_ref[...]
                    + wb_ref[...][:, 0:1] * yb_ref[...])


def _dispatch_call(flat, sa, sb):
    fn = pl.kernel(
        _dispatch_body,
        out_type=jax.ShapeDtypeStruct((NPAD, D_MODEL), jnp.float32),
        mesh=_sc_mesh(),
        scratch_types=[pltpu.VMEM((TPW, D_MODEL), jnp.float32),
                       pltpu.VMEM((1, TPW), jnp.int32),
                       pltpu.VMEM((1, TPW), jnp.int32),
                       pltpu.SemaphoreType.DMA,
                       pltpu.SemaphoreType.DMA],
    )
    return fn(flat, sa, sb)


def _combine_call(yw, sa, sb):
    fn = pl.kernel(
        _combine_body,
        out_type=[jax.ShapeDtypeStruct((SEQ, D_MODEL), jnp.float32),
                  jax.ShapeDtypeStruct((SEQ, D_MODEL), jnp.float32)],
        mesh=_sc_mesh(),
        scratch_types=[pltpu.VMEM((1, TPW), jnp.int32),
                       pltpu.VMEM((1, TPW), jnp.int32),
                       pltpu.VMEM((TPW, D_MODEL), jnp.float32),
                       pltpu.VMEM((TPW, D_MODEL), jnp.float32),
                       pltpu.SemaphoreType.DMA,
                       pltpu.SemaphoreType.DMA,
                       pltpu.SemaphoreType.DMA,
                       pltpu.SemaphoreType.DMA],
    )
    return fn(yw, sa, sb)


def kernel(hidden_states, Wr, W1, W2, W3):
    batch, seq, hidden = hidden_states.shape
    flat = hidden_states.reshape(-1, hidden)

    rwa, rwb, slots2, cnts = pl.pallas_call(
        _router_body,
        out_shape=[jax.ShapeDtypeStruct((SEQ, 128), jnp.float32),
                   jax.ShapeDtypeStruct((SEQ, 128), jnp.float32),
                   jax.ShapeDtypeStruct((SEQ, TOP_K), jnp.int32),
                   jax.ShapeDtypeStruct((1, NUM_EXPERTS), jnp.float32)],
    )(flat, Wr)

    slots_flat = slots2.reshape(NPAIR)
    sa = slots2[:, 0]
    sb = slots2[:, 1]

    padded = (jnp.ceil(cnts[0] / BT) * BT).astype(jnp.int32)
    pstart = (jnp.concatenate([jnp.zeros((1,), jnp.int32),
                               jnp.cumsum(padded)[:-1]]) // BT)
    be = jnp.clip(
        (jnp.arange(NBLK, dtype=jnp.int32)[:, None] >= pstart[None, :])
        .sum(axis=1).astype(jnp.int32) - 1, 0, NUM_EXPERTS - 1)
    first = jnp.concatenate([jnp.ones((1,), jnp.int32),
                             (be[1:] != be[:-1]).astype(jnp.int32)])
    gi = jnp.cumsum(first) - 1
    ngroups = gi[-1] + 1
    gexp = jnp.zeros((NBLK + 1,), jnp.int32).at[gi].set(be)
    nx = gexp[jnp.clip(gi + 1, 0, NBLK)]
    fetch = (first * (gi + 1 < ngroups)).astype(jnp.int32)
    nact = (jnp.sum(padded) // BT).astype(jnp.int32).reshape(1)

    xs = _dispatch_call(flat, sa, sb)

    yw = pl.pallas_call(
        _moe_body,
        grid_spec=pltpu.PrefetchScalarGridSpec(
            num_scalar_prefetch=5,
            grid=(NBLK,),
            in_specs=[
                pl.BlockSpec((BT, D_MODEL), lambda i, *_: (i, 0)),
                pl.BlockSpec(memory_space=pl.ANY),
                pl.BlockSpec(memory_space=pl.ANY),
                pl.BlockSpec(memory_space=pl.ANY),
            ],
            out_specs=pl.BlockSpec((BT, D_MODEL), lambda i, *_: (i, 0)),
            scratch_shapes=[
                pltpu.VMEM((2, D_MODEL, D_FF), jnp.float32),
                pltpu.VMEM((2, D_MODEL, D_FF), jnp.float32),
                pltpu.VMEM((2, D_FF, D_MODEL), jnp.float32),
                pltpu.SemaphoreType.DMA((2, 3)),
            ],
        ),
        out_shape=jax.ShapeDtypeStruct((NPAD, D_MODEL), jnp.float32),
    )(be, gi.astype(jnp.int32), fetch, nx.astype(jnp.int32), nact,
      xs, W1, W3, W2)

    ya, yb = _combine_call(yw, sa, sb)

    out = pl.pallas_call(
        _finish_body,
        grid=(SEQ // 128,),
        in_specs=[pl.BlockSpec((128, D_MODEL), lambda i: (i, 0)),
                  pl.BlockSpec((128, D_MODEL), lambda i: (i, 0)),
                  pl.BlockSpec((128, 128), lambda i: (i, 0)),
                  pl.BlockSpec((128, 128), lambda i: (i, 0))],
        out_specs=pl.BlockSpec((128, D_MODEL), lambda i: (i, 0)),
        out_shape=jax.ShapeDtypeStruct((SEQ, D_MODEL), jnp.float32),
    )(ya, yb, rwa, rwb)
    return out.reshape(batch, seq, hidden)
